# baseline (speedup 1.0000x reference)
import functools

import jax
import jax.numpy as jnp
from jax import lax
from jax.experimental import pallas as pl
from jax.experimental.layout import Layout
from jax.experimental.pallas import tpu as pltpu
from jax.experimental.pallas import tpu_sc as plsc

D = 512
T = 50
TP = 56
NC = 2
NS = 16
NW = NC * NS
B = 4096
NCHUNK = B // NW
NBUF = 4


def _emb_body(table_hbm, idx_hbm, out_hbm, idx_v,
              buf0, buf1, buf2, buf3,
              gsem0, gsem1, gsem2, gsem3,
              ssem0, ssem1, ssem2, ssem3):
  wid = lax.axis_index("s") * NC + lax.axis_index("c")
  base = wid * NCHUNK

  pltpu.sync_copy(idx_hbm.at[pl.ds(base * TP, NCHUNK * TP)], idx_v)

  bufs = (buf0, buf1, buf2, buf3)
  gsems = (gsem0, gsem1, gsem2, gsem3)
  ssems = (ssem0, ssem1, ssem2, ssem3)

  def start_gather(j, b):
    pltpu.async_copy(
        table_hbm.at[idx_v.at[pl.ds(j * TP, T)]], bufs[b], gsems[b])

  def wait_gather(j, b):
    pltpu.make_async_copy(
        table_hbm.at[idx_v.at[pl.ds(j * TP, T)]], bufs[b], gsems[b]).wait()

  def start_scatter(j, b):
    pltpu.async_copy(bufs[b], out_hbm.at[base + j], ssems[b])

  def wait_scatter(j, b):
    pltpu.make_async_copy(bufs[b], out_hbm.at[base + j], ssems[b]).wait()

  for b in range(NBUF):
    start_gather(b, b)

  def body(i, carry):
    j0 = i * NBUF
    for b in range(NBUF):
      j = j0 + b
      wait_gather(j, b)
      start_scatter(j, b)
    for b in range(NBUF):
      j = j0 + b

      @pl.when(j + NBUF < NCHUNK)
      def _():
        wait_scatter(j, b)
        start_gather(j + NBUF, b)

    return carry

  lax.fori_loop(0, NCHUNK // NBUF, body, 0)

  for b in range(NBUF):
    wait_scatter(NCHUNK - NBUF + b, b)


@functools.partial(jax.jit, static_argnames=())
def kernel(tokens, kernel):
  table = kernel
  b, t = tokens.shape
  idx = tokens.astype(jnp.int32)
  idx = jnp.pad(idx, ((0, 0), (0, TP - T))).reshape(-1)

  mesh = plsc.VectorSubcoreMesh(core_axis_name="c", subcore_axis_name="s")
  emb = pl.kernel(
      _emb_body,
      mesh=mesh,
      out_type=jax.ShapeDtypeStruct((B, T, D), jnp.float32),
      compiler_params=pltpu.CompilerParams(use_tc_tiling_on_sc=False),
      scratch_types=(
          [pltpu.VMEM((NCHUNK * TP,), jnp.int32)]
          + [pltpu.VMEM((T, D), jnp.float32) for _ in range(NBUF)]
          + [pltpu.SemaphoreType.DMA for _ in range(2 * NBUF)]
      ),
  )
  out = emb(table, idx)
  return jax.experimental.layout.with_layout_constraint(
      out, Layout(major_to_minor=(0, 1, 2), tiling=()))


# speedup vs baseline: 1.2872x; 1.2872x over previous
import functools

import jax
import jax.numpy as jnp
from jax import lax
from jax.experimental import pallas as pl
from jax.experimental.pallas import tpu as pltpu
from jax.experimental.pallas import tpu_sc as plsc

D = 512
T = 50
TP = 56
NC = 2
NS = 16
NW = NC * NS
B = 4096
NCHUNK = B // NW
NBUF = 4


def _emb_body(table_hbm, idx_hbm, out_hbm, idx_v, table_sp,
              buf0, buf1, buf2, buf3,
              gsem0, gsem1, gsem2, gsem3,
              ssem0, ssem1, ssem2, ssem3):
  sid = lax.axis_index("s")
  wid = sid * NC + lax.axis_index("c")
  base = wid * NCHUNK

  @pl.when(sid == 0)
  def _():
    pltpu.sync_copy(table_hbm, table_sp)

  pltpu.sync_copy(idx_hbm.at[pl.ds(base * TP, NCHUNK * TP)], idx_v)
  plsc.subcore_barrier()

  bufs = (buf0, buf1, buf2, buf3)
  gsems = (gsem0, gsem1, gsem2, gsem3)
  ssems = (ssem0, ssem1, ssem2, ssem3)

  def start_gather(j, b):
    pltpu.async_copy(
        table_sp.at[idx_v.at[pl.ds(j * TP, T)]], bufs[b], gsems[b])

  def wait_gather(j, b):
    pltpu.make_async_copy(
        table_sp.at[idx_v.at[pl.ds(j * TP, T)]], bufs[b], gsems[b]).wait()

  def start_scatter(j, b):
    pltpu.async_copy(bufs[b], out_hbm.at[base + j], ssems[b])

  def wait_scatter(j, b):
    pltpu.make_async_copy(bufs[b], out_hbm.at[base + j], ssems[b]).wait()

  for b in range(NBUF):
    start_gather(b, b)

  def body(i, carry):
    j0 = i * NBUF
    for b in range(NBUF):
      j = j0 + b
      wait_gather(j, b)
      start_scatter(j, b)
    for b in range(NBUF):
      j = j0 + b

      @pl.when(j + NBUF < NCHUNK)
      def _():
        wait_scatter(j, b)
        start_gather(j + NBUF, b)

    return carry

  lax.fori_loop(0, NCHUNK // NBUF, body, 0)

  for b in range(NBUF):
    wait_scatter(NCHUNK - NBUF + b, b)


@functools.partial(jax.jit, static_argnames=())
def kernel(tokens, kernel):
  table = kernel
  b, t = tokens.shape
  idx = tokens.astype(jnp.int32)
  idx = jnp.pad(idx, ((0, 0), (0, TP - T))).reshape(-1)

  mesh = plsc.VectorSubcoreMesh(core_axis_name="c", subcore_axis_name="s")
  emb = pl.kernel(
      _emb_body,
      mesh=mesh,
      out_type=jax.ShapeDtypeStruct((B, T, D), jnp.float32),
      compiler_params=pltpu.CompilerParams(use_tc_tiling_on_sc=False),
      scratch_types=(
          [pltpu.VMEM((NCHUNK * TP,), jnp.int32)]
          + [pltpu.VMEM_SHARED((300, D), jnp.float32)]
          + [pltpu.VMEM((T, D), jnp.float32) for _ in range(NBUF)]
          + [pltpu.SemaphoreType.DMA for _ in range(2 * NBUF)]
      ),
  )
  return emb(table, idx)
